# baseline (speedup 1.0000x reference)
import functools

import jax
import jax.numpy as jnp
from jax import lax
from jax.experimental import pallas as pl
from jax.experimental.pallas import tpu as pltpu
from jax.experimental.pallas import tpu_sc as plsc

N_USERS = 50000
N_ITEMS = 50000
N_ALL = 100000
D = 100
DP = 128
NF = 4
E = 400000
P3 = 300000
P5 = 300000
NNZ = 400000

NC = 2
NS = 16
NW = NC * NS

QS = 9472
NPASS = 3
NROWS_PAD = 2 * NPASS * QS
ACC_R = QS + 16
TRASH = QS
SPAN = QS // NS

G3_MAIN, G3_LAST = 3128, 100000 - 31 * 3128
G5_MAIN, G5_LAST = 1880, 60000 - 31 * 1880

SB = 4992
EPT = E // NS
NSB = EPT // SB
TAIL = EPT - NSB * SB

CCAP = ((EPT + 127) // 128) * 128

_mesh = plsc.VectorSubcoreMesh(core_axis_name="c", subcore_axis_name="s",
                               num_cores=NC, num_subcores=NS)

_i16 = functools.partial(jnp.arange, dtype=jnp.int32)


def _splat(v):
    return jnp.full((16,), v, dtype=jnp.int32)


def _zero_gb(gb, rows):
    z = jnp.zeros((16,), dtype=jnp.float32)

    def body(i, _):
        for l in range(DP // 16):
            gb[i, pl.ds(16 * l, 16)] = z
        return 0

    lax.fori_loop(0, rows, body, 0)


def _zero_acc_span(gb, acc, sid):
    base = SPAN * sid
    for k in range(4):
        pltpu.sync_copy(gb, acc.at[pl.ds(base + 128 * k, 128)])
    pltpu.sync_copy(gb.at[pl.ds(0, 80)], acc.at[pl.ds(base + 512, 80)])


def _readback(acc, out_hbm, sid, qbase):
    base = SPAN * sid
    for k in range(4):
        pltpu.sync_copy(acc.at[pl.ds(base + 128 * k, 128)],
                        out_hbm.at[pl.ds(qbase + base + 128 * k, 128)])
    pltpu.sync_copy(acc.at[pl.ds(base + 512, 80)],
                    out_hbm.at[pl.ds(qbase + base + 512, 80)])


def _paths_body(tbl, p3, p5, psum, idxb, gb, ob):
    cid = lax.axis_index("c")
    sid = lax.axis_index("s")
    wid = cid * NS + sid
    last = wid == NW - 1

    @pl.when(jnp.logical_not(last))
    def _():
        pltpu.sync_copy(p3.at[pl.ds(3 * G3_MAIN * wid, 3 * G3_MAIN)],
                        idxb.at[pl.ds(0, 3 * G3_MAIN)])

    @pl.when(last)
    def _():
        pltpu.sync_copy(p3.at[pl.ds(3 * G3_MAIN * (NW - 1), 3 * G3_LAST)],
                        idxb.at[pl.ds(0, 3 * G3_LAST)])

    n3 = jnp.where(last, G3_LAST // 8, G3_MAIN // 8)
    row3 = G3_MAIN * wid

    def body3(c, _):
        off = pl.multiple_of(c * 24, 24)
        pltpu.sync_copy(tbl.at[idxb.at[pl.ds(off, 24)]], gb.at[pl.ds(0, 24)])
        for g in range(8):
            for l in range(DP // 16):
                s = pl.ds(16 * l, 16)
                ob[g, s] = (gb[3 * g, s] + gb[3 * g + 1, s]) + gb[3 * g + 2, s]
        pltpu.sync_copy(ob, psum.at[pl.ds(row3 + 8 * c, 8)])
        return 0

    lax.fori_loop(0, n3, body3, 0)

    @pl.when(jnp.logical_not(last))
    def _():
        pltpu.sync_copy(p5.at[pl.ds(5 * G5_MAIN * wid, 5 * G5_MAIN)],
                        idxb.at[pl.ds(0, 5 * G5_MAIN)])

    @pl.when(last)
    def _():
        pltpu.sync_copy(p5.at[pl.ds(5 * G5_MAIN * (NW - 1), 5 * G5_LAST)],
                        idxb.at[pl.ds(0, 5 * G5_LAST)])

    n5 = jnp.where(last, G5_LAST // 8, G5_MAIN // 8)
    row5 = 100000 + G5_MAIN * wid

    def body5(c, _):
        off = pl.multiple_of(c * 40, 40)
        pltpu.sync_copy(tbl.at[idxb.at[pl.ds(off, 40)]], gb)
        for g in range(8):
            for l in range(DP // 16):
                s = pl.ds(16 * l, 16)
                ob[g, s] = (((gb[5 * g, s] + gb[5 * g + 1, s])
                             + (gb[5 * g + 2, s] + gb[5 * g + 3, s]))
                            + gb[5 * g + 4, s])
        pltpu.sync_copy(ob, psum.at[pl.ds(row5 + 8 * c, 8)])
        return 0

    lax.fori_loop(0, n5, body5, 0)


_paths_kernel = functools.partial(
    pl.kernel, _paths_body, mesh=_mesh,
    out_type=jax.ShapeDtypeStruct((160000, DP), jnp.float32),
    scratch_types=[
        pltpu.VMEM((5 * G5_MAIN,), jnp.int32),
        pltpu.VMEM((40, DP), jnp.float32),
        pltpu.VMEM((8, DP), jnp.float32),
    ],
    compiler_params=pltpu.CompilerParams(needs_layout_passes=False),
)()


def _kg_body(tbl, heads, tails, isum, hstage, tstage, psel, tbuf, dring, gb,
             acc):
    cid = lax.axis_index("c")
    sid = lax.axis_index("s")
    wid = cid * NS + sid
    eb = EPT * sid
    iota = _i16(16)

    _zero_gb(gb, 128)

    for p in range(NPASS):
        q = NPASS * cid + p
        lo = QS * q

        _zero_acc_span(gb, acc, sid)
        plsc.subcore_barrier()

        def filt_body(i, n, base=0):
            o = i * 16
            h = hstage[pl.ds(o, 16)]
            t = tstage[pl.ds(o, 16)]
            d = h - (N_USERS + lo)
            m = jnp.logical_and(d >= 0, d < QS)
            ones = jnp.where(m, 1, 0).astype(jnp.int32)
            cs = plsc.cumsum(ones)
            pos = n + cs - 1
            plsc.store_scatter(psel, [pos], t | (d << 17), mask=m)
            return n + jnp.max(cs)

        n = jnp.int32(0)
        for sb in range(NSB):
            pltpu.sync_copy(heads.at[pl.ds(eb + SB * sb, SB)], hstage)
            pltpu.sync_copy(tails.at[pl.ds(eb + SB * sb, SB)], tstage)
            n = lax.fori_loop(0, SB // 16, filt_body, n)
        pltpu.sync_copy(heads.at[pl.ds(eb + NSB * SB, TAIL)],
                        hstage.at[pl.ds(0, TAIL)])
        pltpu.sync_copy(tails.at[pl.ds(eb + NSB * SB, TAIL)],
                        tstage.at[pl.ds(0, TAIL)])
        hv = hstage[pl.ds(32, 16)]
        hstage[pl.ds(32, 16)] = jnp.where(iota < 8, hv, -1)
        n = lax.fori_loop(0, 3, filt_body, n)

        nc = (n + 127) // 128

        def pad_body(j, _):
            pv = n + 16 * j + iota
            pm = pv < nc * 128
            pad = (wid * 128 + (pv & 127)) | (TRASH << 17)
            plsc.store_scatter(psel, [pv], pad, mask=pm)
            return 0

        lax.fori_loop(0, 8, pad_body, 0)

        def gs_body(c, _):
            off = pl.multiple_of(c * 128, 128)
            for j in range(8):
                pk = psel[pl.ds(off + 16 * j, 16)]
                tbuf[pl.ds(16 * j, 16)] = pk & 0x1FFFF
                dring[0, pl.ds(16 * j, 16)] = pk >> 17
            pltpu.sync_copy(tbl.at[tbuf], gb)
            pltpu.sync_copy(gb, acc.at[dring.at[0]], add=True)
            return 0

        lax.fori_loop(0, nc, gs_body, 0)
        plsc.subcore_barrier()

        _readback(acc, isum, sid, QS * q)
        plsc.subcore_barrier()
        if p < NPASS - 1:
            _zero_gb(gb, 128)


_kg_kernel = functools.partial(
    pl.kernel, _kg_body, mesh=_mesh,
    out_type=jax.ShapeDtypeStruct((NROWS_PAD, DP), jnp.float32),
    scratch_types=[
        pltpu.VMEM((SB,), jnp.int32),
        pltpu.VMEM((SB,), jnp.int32),
        pltpu.VMEM((CCAP,), jnp.int32),
        pltpu.VMEM((128,), jnp.int32),
        pltpu.VMEM((1, 128), jnp.int32),
        pltpu.VMEM((128, DP), jnp.float32),
        pltpu.VMEM_SHARED((ACC_R, DP), jnp.float32),
    ],
    compiler_params=pltpu.CompilerParams(needs_layout_passes=False),
)()


def _mm_body(itbl, rows, cols, vals, usum, rstage, cbuf, vbuf, gb, dring, acc):
    cid = lax.axis_index("c")
    sid = lax.axis_index("s")
    eb = EPT * sid
    iota = _i16(16)
    big = jnp.int32(1 << 30)

    pltpu.sync_copy(rows.at[pl.ds(eb, EPT)], rstage.at[pl.ds(0, EPT)])
    rv = rstage[pl.ds(EPT - 8, 16)]
    rstage[pl.ds(EPT - 8, 16)] = jnp.where(iota < 8, rv, big)
    for k in range((CCAP - EPT - 8) // 16):
        rstage[pl.ds(EPT + 8 + 16 * k, 16)] = jnp.full((16,), big)

    for p in range(NPASS):
        q = NPASS * cid + p
        lo = QS * q
        hi = lo + QS

        _zero_gb(gb, 128)
        _zero_acc_span(gb, acc, sid)
        plsc.subcore_barrier()

        def cnt_body(i, ab):
            a, b = ab
            r = rstage[pl.ds(i * 16, 16)]
            a = a + jnp.sum(jnp.where(r < lo, 1, 0).astype(jnp.int32))
            b = b + jnp.sum(jnp.where(r < hi, 1, 0).astype(jnp.int32))
            return (a, b)

        a, b = lax.fori_loop(0, CCAP // 16, cnt_body,
                             (jnp.int32(0), jnp.int32(0)))
        k0 = a // 128
        k1 = (b + 127) // 128

        def chunk_body(c, _):
            off = pl.multiple_of(c * 128, 128)
            pltpu.sync_copy(cols.at[pl.ds(eb + off, 128)], cbuf)
            pltpu.sync_copy(vals.at[pl.ds(eb + off, 128)], vbuf)
            pltpu.sync_copy(itbl.at[cbuf], gb)
            for sub in range(8):
                e0 = off + 16 * sub
                rr = rstage[pl.ds(e0, 16)]
                vv = vbuf[pl.ds(16 * sub, 16)]
                ev = e0 + iota
                m = jnp.logical_and(ev >= a, ev < b)
                dv = jnp.where(m, rr - lo, TRASH)
                dring[0, pl.ds(16 * sub, 16)] = dv
                ridx = 16 * sub + iota

                def scale_body(l, _):
                    li = _splat(l)
                    x = plsc.load_gather(gb, [ridx, li])
                    plsc.store_scatter(gb, [ridx, li], x * vv)
                    return 0

                lax.fori_loop(0, DP, scale_body, 0, unroll=8)
            pltpu.sync_copy(gb, acc.at[dring.at[0]], add=True)
            return 0

        lax.fori_loop(k0, k1, chunk_body, 0)
        plsc.subcore_barrier()

        _readback(acc, usum, sid, QS * q)
        plsc.subcore_barrier()


_mm_kernel = functools.partial(
    pl.kernel, _mm_body, mesh=_mesh,
    out_type=jax.ShapeDtypeStruct((NROWS_PAD, DP), jnp.float32),
    scratch_types=[
        pltpu.VMEM((CCAP,), jnp.int32),
        pltpu.VMEM((128,), jnp.int32),
        pltpu.VMEM((128,), jnp.float32),
        pltpu.VMEM((128, DP), jnp.float32),
        pltpu.VMEM((1, 128), jnp.int32),
        pltpu.VMEM_SHARED((ACC_R, DP), jnp.float32),
    ],
    compiler_params=pltpu.CompilerParams(needs_layout_passes=False),
)()


_TC1_R = 800
_TC1_STEPS = 160000 // _TC1_R
_P3_STEPS = 100000 // _TC1_R


def _tc1_body(ps_ref, le_ref, out_ref, acc_ref):
    i = pl.program_id(0)
    sc = jnp.where(i < _P3_STEPS, 1.0 / 3.0, 1.0 / 5.0).astype(jnp.float32)
    x = ps_ref[:, :D] * sc
    s = lax.dot_general(x, le_ref[...], (((1,), (1,)), ((), ())),
                        precision=lax.Precision.HIGHEST)
    s = s * 200.0
    s = s - jnp.max(s, axis=1, keepdims=True)
    e = jnp.exp(s)
    w = e / jnp.sum(e, axis=1, keepdims=True)
    la = lax.dot_general(w, x, (((0,), (0,)), ((), ())),
                         precision=lax.Precision.HIGHEST)

    @pl.when(i == 0)
    def _():
        acc_ref[...] = jnp.zeros_like(acc_ref)

    acc_ref[...] += la

    @pl.when(i == _TC1_STEPS - 1)
    def _():
        out_ref[...] = acc_ref[...]


def _tc1(psum, latent_emb):
    return pl.pallas_call(
        _tc1_body,
        grid=(_TC1_STEPS,),
        in_specs=[
            pl.BlockSpec((_TC1_R, DP), lambda i: (i, 0)),
            pl.BlockSpec((NF, D), lambda i: (0, 0)),
        ],
        out_specs=pl.BlockSpec((NF, D), lambda i: (0, 0)),
        out_shape=jax.ShapeDtypeStruct((NF, D), jnp.float32),
        scratch_shapes=[pltpu.VMEM((NF, D), jnp.float32)],
    )(psum, latent_emb)


_TC2_R = 1000
_TC2_STEPS = N_USERS // _TC2_R


def _tc2_body(ue_ref, us_ref, is_ref, le_ref, w_ref, da_ref, la_ref,
              ua_ref, ia_ref):
    da = da_ref[...]
    da = da - jnp.max(da, axis=1, keepdims=True)
    ea = jnp.exp(da)
    sa = ea / jnp.sum(ea, axis=1, keepdims=True)
    dw = lax.dot_general(sa, w_ref[...], (((1,), (0,)), ((), ())),
                         precision=lax.Precision.HIGHEST)
    dw = 0.4 * dw + 0.6 * la_ref[...]
    nrm = jnp.sqrt(jnp.sum(dw * dw, axis=1, keepdims=True))
    dw = dw / jnp.maximum(nrm, 1e-12)

    s = lax.dot_general(ue_ref[...], le_ref[...], (((1,), (1,)), ((), ())),
                        precision=lax.Precision.HIGHEST)
    s = s - jnp.max(s, axis=1, keepdims=True)
    es = jnp.exp(s)
    w = es / jnp.sum(es, axis=1, keepdims=True)
    f = 1.0 + lax.dot_general(w, dw, (((1,), (0,)), ((), ())),
                              precision=lax.Precision.HIGHEST)
    ua_ref[...] = us_ref[:, :D] * f

    cnt = is_ref[:, D:D + 1]
    ia_ref[...] = is_ref[:, :D] / jnp.maximum(cnt, 1.0)


def _tc2(user_emb, usum, isum, latent_emb, weight, datt, lagg):
    return pl.pallas_call(
        _tc2_body,
        grid=(_TC2_STEPS,),
        in_specs=[
            pl.BlockSpec((_TC2_R, D), lambda i: (i, 0)),
            pl.BlockSpec((_TC2_R, DP), lambda i: (i, 0)),
            pl.BlockSpec((_TC2_R, DP), lambda i: (i, 0)),
            pl.BlockSpec((NF, D), lambda i: (0, 0)),
            pl.BlockSpec((NF, D), lambda i: (0, 0)),
            pl.BlockSpec((NF, NF), lambda i: (0, 0)),
            pl.BlockSpec((NF, D), lambda i: (0, 0)),
        ],
        out_specs=[
            pl.BlockSpec((_TC2_R, D), lambda i: (i, 0)),
            pl.BlockSpec((_TC2_R, D), lambda i: (i, 0)),
        ],
        out_shape=[
            jax.ShapeDtypeStruct((N_USERS, D), jnp.float32),
            jax.ShapeDtypeStruct((N_ITEMS, D), jnp.float32),
        ],
    )(user_emb, usum, isum, latent_emb, weight, datt, lagg)


def kernel(user_emb, item_emb, all_embed, latent_emb, heads_tensor,
           tails_tensor, mm_rows, mm_cols, mm_vals, weight,
           disen_weight_att, path_nodes_3, path_nodes_5):
    tbl = jnp.concatenate(
        [all_embed, jnp.ones((N_ALL, 1), jnp.float32),
         jnp.zeros((N_ALL, DP - D - 1), jnp.float32)], axis=1)
    itbl = jnp.concatenate(
        [item_emb, jnp.zeros((N_ITEMS, DP - D), jnp.float32)], axis=1)
    p3 = path_nodes_3.astype(jnp.int32)
    p5 = path_nodes_5.astype(jnp.int32)
    heads = heads_tensor.astype(jnp.int32)
    tails = tails_tensor.astype(jnp.int32)
    rows = mm_rows.astype(jnp.int32)
    cols = jnp.concatenate(
        [mm_cols.astype(jnp.int32), jnp.zeros((CCAP - EPT,), jnp.int32)])
    vals = jnp.concatenate(
        [mm_vals, jnp.zeros((CCAP - EPT,), jnp.float32)])

    psum = _paths_kernel(tbl, p3, p5)
    isum = _kg_kernel(tbl, heads, tails)
    usum = _mm_kernel(itbl, rows, cols, vals)

    lagg = _tc1(psum, latent_emb)
    user_agg, item_agg = _tc2(user_emb, usum, isum, latent_emb, weight,
                              disen_weight_att, lagg)
    return (user_agg, item_agg)


# speedup vs baseline: 1.4429x; 1.4429x over previous
import functools

import jax
import jax.numpy as jnp
from jax import lax
from jax.experimental import pallas as pl
from jax.experimental.pallas import tpu as pltpu
from jax.experimental.pallas import tpu_sc as plsc

N_USERS = 50000
N_ITEMS = 50000
N_ALL = 100000
D = 100
DP = 128
NF = 4
E = 400000
P3 = 300000
P5 = 300000
NNZ = 400000

NC = 2
NS = 16
NW = NC * NS

QS = 8704
NPASS = 3
NROWS_PAD = 2 * NPASS * QS
ACC_R = QS + 16
TRASH = QS
SPAN = QS // NS
NBUF = 4
CH = 48

G3_MAIN, G3_LAST = 3128, 100000 - 31 * 3128
G5_MAIN, G5_LAST = 1880, 60000 - 31 * 1880

SB = 2496
EPT = E // NS
NSB = EPT // SB
TAIL = EPT - NSB * SB

CCAP = ((EPT + 127) // 128) * 128

_mesh = plsc.VectorSubcoreMesh(core_axis_name="c", subcore_axis_name="s",
                               num_cores=NC, num_subcores=NS)

_i16 = functools.partial(jnp.arange, dtype=jnp.int32)


def _splat(v):
    return jnp.full((16,), v, dtype=jnp.int32)


def _zero_gb(gb3, rows):
    z = jnp.zeros((16,), dtype=jnp.float32)

    def body(i, _):
        for l in range(DP // 16):
            gb3[0, i, pl.ds(16 * l, 16)] = z
        return 0

    lax.fori_loop(0, rows, body, 0)


def _zero_acc_span(zb, acc, sid):
    base = SPAN * sid
    for k in range(SPAN // CH):
        pltpu.sync_copy(zb, acc.at[pl.ds(base + CH * k, CH)])
    rem = SPAN - (SPAN // CH) * CH
    if rem:
        pltpu.sync_copy(zb.at[pl.ds(0, rem)],
                        acc.at[pl.ds(base + (SPAN // CH) * CH, rem)])


def _readback(acc, out_hbm, sid, qbase):
    base = SPAN * sid
    for k in range(4):
        pltpu.sync_copy(acc.at[pl.ds(base + 128 * k, 128)],
                        out_hbm.at[pl.ds(qbase + base + 128 * k, 128)])
    pltpu.sync_copy(acc.at[pl.ds(base + 512, 32)],
                    out_hbm.at[pl.ds(qbase + base + 512, 32)])


def _paths_body(tbl, p3, p5, psum, idxb, gb3, gb5, ob, gsem, osem):
    cid = lax.axis_index("c")
    sid = lax.axis_index("s")
    wid = cid * NS + sid
    last = wid == NW - 1

    def phase(src, gmain, glast, grp, row_off, gbuf):
        st = 8 * grp

        @pl.when(jnp.logical_not(last))
        def _():
            pltpu.sync_copy(src.at[pl.ds(st * (gmain // 8) * wid, st * (gmain // 8))],
                            idxb.at[pl.ds(0, st * (gmain // 8))])

        @pl.when(last)
        def _():
            pltpu.sync_copy(src.at[pl.ds(st * (gmain // 8) * (NW - 1), st * (glast // 8))],
                            idxb.at[pl.ds(0, st * (glast // 8))])

        n = jnp.where(last, glast // 8, gmain // 8)
        row_base = row_off + gmain * wid

        def start_gather(c, b):
            off = pl.multiple_of(c * st, 8)
            pltpu.async_copy(tbl.at[idxb.at[pl.ds(off, st)]], gbuf.at[b],
                             gsem.at[b])

        def wait_gather(c, b):
            off = pl.multiple_of(c * st, 8)
            pltpu.make_async_copy(tbl.at[idxb.at[pl.ds(off, st)]],
                                  gbuf.at[b], gsem.at[b]).wait()

        def wait_write(b):
            pltpu.make_async_copy(ob.at[b], psum.at[pl.ds(row_base, 8)],
                                  osem.at[b]).wait()

        for b in range(2):
            @pl.when(b < n)
            def _(b=b):
                start_gather(b, b)

        def lbody(c2, _):
            for b in range(NBUF):
                c = c2 * NBUF + b

                @pl.when(c < n)
                def _(c=c, b=b):
                    wait_gather(c, b)

                    @pl.when(c >= NBUF)
                    def _():
                        wait_write(b)

                    for g in range(8):
                        for l in range(DP // 16):
                            s = pl.ds(16 * l, 16)
                            acc = gbuf[b, grp * g, s] + gbuf[b, grp * g + 1, s]
                            for j in range(2, grp):
                                acc = acc + gbuf[b, grp * g + j, s]
                            ob[b, g, s] = acc
                    pltpu.async_copy(ob.at[b], psum.at[pl.ds(row_base + 8 * c, 8)],
                                     osem.at[b])

                    @pl.when(c + 2 < n)
                    def _():
                        start_gather(c + 2, (b + 2) % NBUF)
            return 0

        lax.fori_loop(0, (n + NBUF - 1) // NBUF, lbody, 0)
        for b in range(NBUF):
            @pl.when(b < n)
            def _(b=b):
                wait_write(b)

    phase(p3, G3_MAIN, G3_LAST, 3, 0, gb3)
    phase(p5, G5_MAIN, G5_LAST, 5, 100000, gb5)


_paths_kernel = functools.partial(
    pl.kernel, _paths_body, mesh=_mesh,
    out_type=jax.ShapeDtypeStruct((160000, DP), jnp.float32),
    scratch_types=[
        pltpu.VMEM((5 * G5_MAIN,), jnp.int32),
        pltpu.VMEM((NBUF, 24, DP), jnp.float32),
        pltpu.VMEM((NBUF, 40, DP), jnp.float32),
        pltpu.VMEM((NBUF, 8, DP), jnp.float32),
        pltpu.SemaphoreType.DMA((NBUF,)),
        pltpu.SemaphoreType.DMA((NBUF,)),
    ],
    compiler_params=pltpu.CompilerParams(needs_layout_passes=False),
)()


def _kg_body(tbl, heads, tails, isum, hstage, tstage, psel, tbuf, dring, gb,
             gsem, ssem, acc):
    cid = lax.axis_index("c")
    sid = lax.axis_index("s")
    wid = cid * NS + sid
    eb = EPT * sid
    iota = _i16(16)

    for p in range(NPASS):
        q = NPASS * cid + p
        lo = QS * q

        _zero_gb(gb, CH)
        _zero_acc_span(gb.at[0], acc, sid)
        plsc.subcore_barrier()

        def filt_body(i, n, base=0):
            o = i * 16
            h = hstage[pl.ds(o, 16)]
            t = tstage[pl.ds(o, 16)]
            d = h - (N_USERS + lo)
            m = jnp.logical_and(d >= 0, d < QS)
            ones = jnp.where(m, 1, 0).astype(jnp.int32)
            cs = plsc.cumsum(ones)
            pos = n + cs - 1
            plsc.store_scatter(psel, [pos], t | (d << 17), mask=m)
            return n + jnp.max(cs)

        n = jnp.int32(0)
        for sb in range(NSB):
            pltpu.sync_copy(heads.at[pl.ds(eb + SB * sb, SB)], hstage)
            pltpu.sync_copy(tails.at[pl.ds(eb + SB * sb, SB)], tstage)
            n = lax.fori_loop(0, SB // 16, filt_body, n)
        pltpu.sync_copy(heads.at[pl.ds(eb + NSB * SB, TAIL)],
                        hstage.at[pl.ds(0, TAIL)])
        pltpu.sync_copy(tails.at[pl.ds(eb + NSB * SB, TAIL)],
                        tstage.at[pl.ds(0, TAIL)])
        hv = hstage[pl.ds(32, 16)]
        hstage[pl.ds(32, 16)] = jnp.where(iota < 8, hv, -1)
        n = lax.fori_loop(0, 3, filt_body, n)

        nc = (n + CH - 1) // CH

        def pad_body(j, _):
            pv = n + 16 * j + iota
            pm = pv < nc * CH
            pad = (wid * 128 + (pv & 127)) | (TRASH << 17)
            plsc.store_scatter(psel, [pv], pad, mask=pm)
            return 0

        lax.fori_loop(0, CH // 16, pad_body, 0)

        def unpack(c, b):
            off = pl.multiple_of(c * CH, 16)
            for j in range(CH // 16):
                pk = psel[pl.ds(off + 16 * j, 16)]
                tbuf[b, pl.ds(16 * j, 16)] = pk & 0x1FFFF
                dring[b, pl.ds(16 * j, 16)] = pk >> 17

        def start_gather(c, b):
            pltpu.async_copy(tbl.at[tbuf.at[b]], gb.at[b], gsem.at[b])

        def wait_gather(b):
            pltpu.make_async_copy(tbl.at[tbuf.at[b]], gb.at[b],
                                  gsem.at[b]).wait()

        def wait_scatter(b):
            pltpu.make_async_copy(gb.at[b], acc.at[dring.at[b]],
                                  ssem.at[b]).wait()

        for b in range(2):
            @pl.when(b < nc)
            def _(b=b):
                unpack(b, b)
                start_gather(b, b)

        def lbody(c2, _):
            for b in range(NBUF):
                c = c2 * NBUF + b

                @pl.when(c < nc)
                def _(c=c, b=b):
                    wait_gather(b)
                    pltpu.async_copy(gb.at[b], acc.at[dring.at[b]],
                                     ssem.at[b], add=True)
                    bb = (b + 2) % NBUF

                    @pl.when(c + 2 < nc)
                    def _():
                        @pl.when(c + 2 >= NBUF)
                        def _():
                            wait_scatter(bb)
                        unpack(c + 2, bb)
                        start_gather(c + 2, bb)
            return 0

        lax.fori_loop(0, (nc + NBUF - 1) // NBUF, lbody, 0)
        for b in range(NBUF):
            @pl.when(b < nc)
            def _(b=b):
                wait_scatter(b)
        plsc.subcore_barrier()

        _readback(acc, isum, sid, QS * q)
        plsc.subcore_barrier()
        if p < NPASS - 1:
            _zero_gb(gb, 128)


_kg_kernel = functools.partial(
    pl.kernel, _kg_body, mesh=_mesh,
    out_type=jax.ShapeDtypeStruct((NROWS_PAD, DP), jnp.float32),
    scratch_types=[
        pltpu.VMEM((SB,), jnp.int32),
        pltpu.VMEM((SB,), jnp.int32),
        pltpu.VMEM((CCAP,), jnp.int32),
        pltpu.VMEM((NBUF, CH), jnp.int32),
        pltpu.VMEM((NBUF, CH), jnp.int32),
        pltpu.VMEM((NBUF, CH, DP), jnp.float32),
        pltpu.SemaphoreType.DMA((NBUF,)),
        pltpu.SemaphoreType.DMA((NBUF,)),
        pltpu.VMEM_SHARED((ACC_R, DP), jnp.float32),
    ],
    compiler_params=pltpu.CompilerParams(needs_layout_passes=False),
)()


def _mm_body(itbl, rows, cols, vals, usum, rstage, cbuf, vbuf, gb, dring,
             gsem, ssem, csem, acc):
    cid = lax.axis_index("c")
    sid = lax.axis_index("s")
    eb = EPT * sid
    iota = _i16(16)
    big = jnp.int32(1 << 30)

    pltpu.sync_copy(rows.at[pl.ds(eb, EPT)], rstage.at[pl.ds(0, EPT)])
    rv = rstage[pl.ds(EPT - 8, 16)]
    rstage[pl.ds(EPT - 8, 16)] = jnp.where(iota < 8, rv, big)
    for k in range((CCAP - EPT - 8) // 16):
        rstage[pl.ds(EPT + 8 + 16 * k, 16)] = jnp.full((16,), big)

    def start_cv(c, b):
        off = pl.multiple_of(c * CH, 8)
        pltpu.async_copy(cols.at[pl.ds(eb + off, CH)], cbuf.at[b], csem.at[b])
        pltpu.async_copy(vals.at[pl.ds(eb + off, CH)], vbuf.at[b], csem.at[b])

    def wait_cv(c, b):
        off = pl.multiple_of(c * CH, 8)
        pltpu.make_async_copy(cols.at[pl.ds(eb + off, CH)], cbuf.at[b],
                              csem.at[b]).wait()
        pltpu.make_async_copy(vals.at[pl.ds(eb + off, CH)], vbuf.at[b],
                              csem.at[b]).wait()

    def start_gather(b):
        pltpu.async_copy(itbl.at[cbuf.at[b]], gb.at[b], gsem.at[b])

    def wait_gather(b):
        pltpu.make_async_copy(itbl.at[cbuf.at[b]], gb.at[b], gsem.at[b]).wait()

    def wait_scatter(b):
        pltpu.make_async_copy(gb.at[b], acc.at[dring.at[b]], ssem.at[b]).wait()

    for p in range(NPASS):
        q = NPASS * cid + p
        lo = QS * q
        hi = lo + QS

        _zero_gb(gb, CH)
        _zero_acc_span(gb.at[0], acc, sid)
        plsc.subcore_barrier()

        def cnt_body(i, ab):
            a, b = ab
            r = rstage[pl.ds(i * 16, 16)]
            a = a + jnp.sum(jnp.where(r < lo, 1, 0).astype(jnp.int32))
            b = b + jnp.sum(jnp.where(r < hi, 1, 0).astype(jnp.int32))
            return (a, b)

        a, b = lax.fori_loop(0, CCAP // 16, cnt_body,
                             (jnp.int32(0), jnp.int32(0)))
        k0 = a // CH
        k1 = (b + CH - 1) // CH
        nc = k1 - k0

        for j in range(3):
            @pl.when(j < nc)
            def _(j=j):
                start_cv(k0 + j, j)
        for j in range(2):
            @pl.when(j < nc)
            def _(j=j):
                wait_cv(k0 + j, j)
                start_gather(j)

        def lbody(c2, _):
            for bslot in range(NBUF):
                ci = c2 * NBUF + bslot

                @pl.when(ci < nc)
                def _(ci=ci, bslot=bslot):
                    c = k0 + ci
                    off = pl.multiple_of(c * CH, 16)
                    wait_gather(bslot)
                    for sub in range(CH // 16):
                        e0 = off + 16 * sub
                        rr = rstage[pl.ds(e0, 16)]
                        vv = vbuf[bslot, pl.ds(16 * sub, 16)]
                        ev = e0 + iota
                        m = jnp.logical_and(ev >= a, ev < b)
                        dv = jnp.where(m, rr - lo, TRASH)
                        dring[bslot, pl.ds(16 * sub, 16)] = dv
                        ridx = 16 * sub + iota

                        def scale_body(l, _):
                            li = _splat(l)
                            x = plsc.load_gather(gb.at[bslot], [ridx, li])
                            plsc.store_scatter(gb.at[bslot], [ridx, li],
                                               x * vv)
                            return 0

                        lax.fori_loop(0, D, scale_body, 0, unroll=4)
                    pltpu.async_copy(gb.at[bslot], acc.at[dring.at[bslot]],
                                     ssem.at[bslot], add=True)
                    b2 = (bslot + 2) % NBUF

                    @pl.when(ci + 2 < nc)
                    def _():
                        @pl.when(ci + 2 >= NBUF)
                        def _():
                            wait_scatter(b2)
                        wait_cv(k0 + ci + 2, b2)
                        start_gather(b2)

                    @pl.when(ci + 3 < nc)
                    def _():
                        start_cv(k0 + ci + 3, (bslot + 3) % NBUF)
            return 0

        lax.fori_loop(0, (nc + NBUF - 1) // NBUF, lbody, 0)
        for bslot in range(NBUF):
            @pl.when(bslot < nc)
            def _(bslot=bslot):
                wait_scatter(bslot)
        plsc.subcore_barrier()

        _readback(acc, usum, sid, QS * q)
        plsc.subcore_barrier()


_mm_kernel = functools.partial(
    pl.kernel, _mm_body, mesh=_mesh,
    out_type=jax.ShapeDtypeStruct((NROWS_PAD, DP), jnp.float32),
    scratch_types=[
        pltpu.VMEM((CCAP,), jnp.int32),
        pltpu.VMEM((NBUF, CH), jnp.int32),
        pltpu.VMEM((NBUF, CH), jnp.float32),
        pltpu.VMEM((NBUF, CH, DP), jnp.float32),
        pltpu.VMEM((NBUF, CH), jnp.int32),
        pltpu.SemaphoreType.DMA((NBUF,)),
        pltpu.SemaphoreType.DMA((NBUF,)),
        pltpu.SemaphoreType.DMA((NBUF,)),
        pltpu.VMEM_SHARED((ACC_R, DP), jnp.float32),
    ],
    compiler_params=pltpu.CompilerParams(needs_layout_passes=False),
)()


_TC1_R = 2000
_TC1_STEPS = 160000 // _TC1_R
_P3_STEPS = 100000 // _TC1_R


def _tc1_body(ps_ref, le_ref, w_ref, da_ref, out_ref, acc_ref):
    i = pl.program_id(0)
    sc = jnp.where(i < _P3_STEPS, 1.0 / 3.0, 1.0 / 5.0).astype(jnp.float32)
    x = ps_ref[:, :D] * sc
    s = lax.dot_general(x, le_ref[...], (((1,), (1,)), ((), ())),
                        precision=lax.Precision.HIGHEST)
    s = s * 200.0
    s = s - jnp.max(s, axis=1, keepdims=True)
    e = jnp.exp(s)
    w = e / jnp.sum(e, axis=1, keepdims=True)
    la = lax.dot_general(w, x, (((0,), (0,)), ((), ())),
                         precision=lax.Precision.HIGHEST)

    @pl.when(i == 0)
    def _():
        acc_ref[...] = jnp.zeros_like(acc_ref)

    acc_ref[...] += la

    @pl.when(i == _TC1_STEPS - 1)
    def _():
        da = da_ref[...]
        da = da - jnp.max(da, axis=1, keepdims=True)
        ea = jnp.exp(da)
        sa = ea / jnp.sum(ea, axis=1, keepdims=True)
        dw = lax.dot_general(sa, w_ref[...], (((1,), (0,)), ((), ())),
                             precision=lax.Precision.HIGHEST)
        dw = 0.4 * dw + 0.6 * acc_ref[...]
        nrm = jnp.sqrt(jnp.sum(dw * dw, axis=1, keepdims=True))
        out_ref[...] = dw / jnp.maximum(nrm, 1e-12)


def _tc1(psum, latent_emb, weight, datt):
    return pl.pallas_call(
        _tc1_body,
        grid=(_TC1_STEPS,),
        in_specs=[
            pl.BlockSpec((_TC1_R, DP), lambda i: (i, 0)),
            pl.BlockSpec((NF, D), lambda i: (0, 0)),
            pl.BlockSpec((NF, D), lambda i: (0, 0)),
            pl.BlockSpec((NF, NF), lambda i: (0, 0)),
        ],
        out_specs=pl.BlockSpec((NF, D), lambda i: (0, 0)),
        out_shape=jax.ShapeDtypeStruct((NF, D), jnp.float32),
        scratch_shapes=[pltpu.VMEM((NF, D), jnp.float32)],
    )(psum, latent_emb, weight, datt)


_TC2_R = 2000
_TC2_STEPS = N_USERS // _TC2_R


def _tc2_body(ue_ref, us_ref, is_ref, le_ref, dw_ref, ua_ref, ia_ref):
    s = lax.dot_general(ue_ref[...], le_ref[...], (((1,), (1,)), ((), ())),
                        precision=lax.Precision.HIGHEST)
    s = s - jnp.max(s, axis=1, keepdims=True)
    es = jnp.exp(s)
    w = es / jnp.sum(es, axis=1, keepdims=True)
    f = 1.0 + lax.dot_general(w, dw_ref[...], (((1,), (0,)), ((), ())),
                              precision=lax.Precision.HIGHEST)
    ua_ref[...] = us_ref[:, :D] * f

    cnt = is_ref[:, D:D + 1]
    ia_ref[...] = is_ref[:, :D] / jnp.maximum(cnt, 1.0)


def _tc2(user_emb, usum, isum, latent_emb, dw):
    return pl.pallas_call(
        _tc2_body,
        grid=(_TC2_STEPS,),
        in_specs=[
            pl.BlockSpec((_TC2_R, D), lambda i: (i, 0)),
            pl.BlockSpec((_TC2_R, DP), lambda i: (i, 0)),
            pl.BlockSpec((_TC2_R, DP), lambda i: (i, 0)),
            pl.BlockSpec((NF, D), lambda i: (0, 0)),
            pl.BlockSpec((NF, D), lambda i: (0, 0)),
        ],
        out_specs=[
            pl.BlockSpec((_TC2_R, D), lambda i: (i, 0)),
            pl.BlockSpec((_TC2_R, D), lambda i: (i, 0)),
        ],
        out_shape=[
            jax.ShapeDtypeStruct((N_USERS, D), jnp.float32),
            jax.ShapeDtypeStruct((N_ITEMS, D), jnp.float32),
        ],
    )(user_emb, usum, isum, latent_emb, dw)


def kernel(user_emb, item_emb, all_embed, latent_emb, heads_tensor,
           tails_tensor, mm_rows, mm_cols, mm_vals, weight,
           disen_weight_att, path_nodes_3, path_nodes_5):
    tbl = jnp.concatenate(
        [all_embed, jnp.ones((N_ALL, 1), jnp.float32),
         jnp.zeros((N_ALL, DP - D - 1), jnp.float32)], axis=1)
    itbl = jnp.concatenate(
        [item_emb, jnp.zeros((N_ITEMS, DP - D), jnp.float32)], axis=1)
    p3 = path_nodes_3.astype(jnp.int32)
    p5 = path_nodes_5.astype(jnp.int32)
    heads = heads_tensor.astype(jnp.int32)
    tails = tails_tensor.astype(jnp.int32)
    rows = mm_rows.astype(jnp.int32)
    cols = jnp.concatenate(
        [mm_cols.astype(jnp.int32), jnp.zeros((CCAP - EPT,), jnp.int32)])
    vals = jnp.concatenate(
        [mm_vals, jnp.zeros((CCAP - EPT,), jnp.float32)])

    psum = _paths_kernel(tbl, p3, p5)
    isum = _kg_kernel(tbl, heads, tails)
    usum = _mm_kernel(itbl, rows, cols, vals)

    dw = _tc1(psum, latent_emb, weight, disen_weight_att)
    user_agg, item_agg = _tc2(user_emb, usum, isum, latent_emb, dw)
    return (user_agg, item_agg)
